# Initial kernel scaffold; baseline (speedup 1.0000x reference)
import jax
import jax.numpy as jnp
from jax.experimental import pallas as pl


def kernel(x, e, W):
    raise NotImplementedError("write your pallas kernel here")


# speedup vs baseline: 1.0988x; 1.0988x over previous
import functools

import jax
import jax.numpy as jnp
from jax import lax
from jax.experimental import pallas as pl
from jax.experimental.pallas import tpu as pltpu
from jax.experimental.pallas import tpu_sc as plsc

N = 65536
D = 32
K = 8192

BN = 256
BK = 4096
NB_N = N // BN
NB_K = K // BK

_NW = 32
_BPW = N // _NW
_CHUNK = 128
_NCH = _BPW // _CHUNK


def _bf16_round(x):
    return x.astype(jnp.bfloat16).astype(jnp.float32)


def _vq_tc_body(et_ref, w_ref, idx_ref, loss_ref, bbf_ref, bexact_ref,
                bidx_ref):
    i = pl.program_id(0)
    j = pl.program_id(1)
    etb = et_ref[...]
    wb = w_ref[...]
    mm = lax.dot_general(wb.astype(jnp.bfloat16), etb.astype(jnp.bfloat16),
                         (((1,), (0,)), ((), ())),
                         preferred_element_type=jnp.float32)
    wn = jnp.sum(wb * wb, axis=1, keepdims=True)
    en = jnp.sum(etb * etb, axis=0, keepdims=True)
    s = (en + wn) - 2.0 * mm
    m = jnp.min(s, axis=0, keepdims=True)
    ids = lax.broadcasted_iota(jnp.int32, s.shape, 0) + j * BK
    amin = jnp.min(jnp.where(s == m, ids, jnp.int32(K)), axis=0,
                   keepdims=True)

    @pl.when(j == 0)
    def _():
        bbf_ref[...] = _bf16_round(m)
        bexact_ref[...] = m
        bidx_ref[...] = amin

    @pl.when(j > 0)
    def _():
        upd = m < bbf_ref[...]
        bbf_ref[...] = jnp.where(upd, _bf16_round(m), bbf_ref[...])
        bexact_ref[...] = jnp.where(upd, m, bexact_ref[...])
        bidx_ref[...] = jnp.where(upd, amin, bidx_ref[...])

    @pl.when(j == NB_K - 1)
    def _():
        idx_ref[...] = bidx_ref[...].reshape(1, 1, BN)
        part = jnp.sum(bexact_ref[...])
        prev = jnp.where(i == 0, jnp.zeros((1, 1), jnp.float32),
                         loss_ref[...])
        acc = prev + part
        scale = jnp.float32(1.25 / (N * D))
        loss_ref[...] = jnp.where(i == NB_N - 1, acc * scale, acc)


def _argmin_and_loss(eT, w):
    return pl.pallas_call(
        _vq_tc_body,
        grid=(NB_N, NB_K),
        in_specs=[
            pl.BlockSpec((D, BN), lambda i, j: (0, i)),
            pl.BlockSpec((BK, D), lambda i, j: (j, 0)),
        ],
        out_specs=[
            pl.BlockSpec((1, 1, BN), lambda i, j: (i, 0, 0)),
            pl.BlockSpec((1, 1), lambda i, j: (0, 0)),
        ],
        out_shape=[
            jax.ShapeDtypeStruct((NB_N, 1, BN), jnp.int32),
            jax.ShapeDtypeStruct((1, 1), jnp.float32),
        ],
        scratch_shapes=[
            pltpu.VMEM((1, BN), jnp.float32),
            pltpu.VMEM((1, BN), jnp.float32),
            pltpu.VMEM((1, BN), jnp.int32),
        ],
        compiler_params=pltpu.CompilerParams(
            dimension_semantics=("arbitrary", "arbitrary"),
        ),
    )(eT, w)


@functools.cache
def _make_sc_gather():
    @functools.partial(
        pl.kernel,
        mesh=plsc.VectorSubcoreMesh(core_axis_name="c", subcore_axis_name="s"),
        out_type=jax.ShapeDtypeStruct((N, D), jnp.float32),
        scratch_types=[
            pltpu.VMEM((_NCH, _CHUNK), jnp.int32),
            pltpu.VMEM((_BPW, D), jnp.float32),
            pltpu.SemaphoreType.DMA,
        ],
        compiler_params=pltpu.CompilerParams(use_tc_tiling_on_sc=False),
    )
    def _sc_gather(w_hbm, idx_hbm, out_hbm, idx_v, rows_v, sem):
        wid = lax.axis_index("s") * 2 + lax.axis_index("c")
        pltpu.sync_copy(idx_hbm.at[wid], idx_v)
        copies = []
        for c in range(_NCH):
            copies.append(pltpu.async_copy(
                w_hbm.at[idx_v.at[c]],
                rows_v.at[pl.ds(c * _CHUNK, _CHUNK)],
                sem))
        for cp in copies:
            cp.wait()
        pltpu.sync_copy(rows_v, out_hbm.at[pl.ds(wid * _BPW, _BPW)])

    return _sc_gather


def kernel(x, e, W):
    del x
    idx3, loss = _argmin_and_loss(e.T, W)
    idx = idx3.reshape(_NW, _NCH, _CHUNK)
    quantized = _make_sc_gather()(W, idx)
    return quantized, loss[0, 0]


# speedup vs baseline: 1.2302x; 1.1196x over previous
import functools

import jax
import jax.numpy as jnp
from jax import lax
from jax.experimental import pallas as pl
from jax.experimental.pallas import tpu as pltpu
from jax.experimental.pallas import tpu_sc as plsc

N = 65536
D = 32
K = 8192

BN = 256
BK = 4096
NB_N = N // BN
NB_K = K // BK

_NW = 32
_BPW = N // _NW
_CHUNK = 128
_NCH = _BPW // _CHUNK


def _bf16_round(x):
    return x.astype(jnp.bfloat16).astype(jnp.float32)


def _vq_tc_body(et_ref, w_ref, idx_ref, loss_ref, bbf_ref, bexact_ref,
                bidx_ref):
    i = pl.program_id(0)
    j = pl.program_id(1)
    etb = et_ref[...]
    wb = w_ref[...]
    mm2 = lax.dot_general((wb + wb).astype(jnp.bfloat16),
                          etb.astype(jnp.bfloat16),
                          (((1,), (0,)), ((), ())),
                          preferred_element_type=jnp.float32)
    wn = jnp.sum(wb * wb, axis=1, keepdims=True)
    en = jnp.sum(etb * etb, axis=0, keepdims=True)
    s = (en + wn) - mm2
    m = jnp.min(s, axis=0, keepdims=True)
    ids = lax.broadcasted_iota(jnp.int32, s.shape, 0)
    amin = jnp.min(jnp.where(s == m, ids, jnp.int32(K)), axis=0,
                   keepdims=True) + j * BK

    @pl.when(j == 0)
    def _():
        bbf_ref[...] = _bf16_round(m)
        bexact_ref[...] = m
        bidx_ref[...] = amin

    @pl.when(j > 0)
    def _():
        upd = m < bbf_ref[...]
        bbf_ref[...] = jnp.where(upd, _bf16_round(m), bbf_ref[...])
        bexact_ref[...] = jnp.where(upd, m, bexact_ref[...])
        bidx_ref[...] = jnp.where(upd, amin, bidx_ref[...])

    @pl.when(j == NB_K - 1)
    def _():
        idx_ref[...] = bidx_ref[...].reshape(1, 1, BN)
        part = jnp.sum(bexact_ref[...])
        prev = jnp.where(i == 0, jnp.zeros((1, 1), jnp.float32),
                         loss_ref[...])
        acc = prev + part
        scale = jnp.float32(1.25 / (N * D))
        loss_ref[...] = jnp.where(i == NB_N - 1, acc * scale, acc)


def _argmin_and_loss(eT, w):
    return pl.pallas_call(
        _vq_tc_body,
        grid=(NB_N, NB_K),
        in_specs=[
            pl.BlockSpec((D, BN), lambda i, j: (0, i)),
            pl.BlockSpec((BK, D), lambda i, j: (j, 0)),
        ],
        out_specs=[
            pl.BlockSpec((1, 1, BN), lambda i, j: (i, 0, 0)),
            pl.BlockSpec((1, 1), lambda i, j: (0, 0)),
        ],
        out_shape=[
            jax.ShapeDtypeStruct((NB_N, 1, BN), jnp.int32),
            jax.ShapeDtypeStruct((1, 1), jnp.float32),
        ],
        scratch_shapes=[
            pltpu.VMEM((1, BN), jnp.float32),
            pltpu.VMEM((1, BN), jnp.float32),
            pltpu.VMEM((1, BN), jnp.int32),
        ],
        compiler_params=pltpu.CompilerParams(
            dimension_semantics=("arbitrary", "arbitrary"),
        ),
    )(eT, w)


@functools.cache
def _make_sc_gather():
    @functools.partial(
        pl.kernel,
        mesh=plsc.VectorSubcoreMesh(core_axis_name="c", subcore_axis_name="s"),
        out_type=jax.ShapeDtypeStruct((N, D), jnp.float32),
        scratch_types=[
            pltpu.VMEM((_NCH, _CHUNK), jnp.int32),
            pltpu.VMEM((_BPW, D), jnp.float32),
            pltpu.SemaphoreType.DMA,
        ],
        compiler_params=pltpu.CompilerParams(use_tc_tiling_on_sc=False),
    )
    def _sc_gather(w_hbm, idx_hbm, out_hbm, idx_v, rows_v, sem):
        wid = lax.axis_index("s") * 2 + lax.axis_index("c")
        pltpu.sync_copy(idx_hbm.at[wid], idx_v)
        copies = []
        for c in range(_NCH):
            copies.append(pltpu.async_copy(
                w_hbm.at[idx_v.at[c]],
                rows_v.at[pl.ds(c * _CHUNK, _CHUNK)],
                sem))
        for cp in copies:
            cp.wait()
        pltpu.sync_copy(rows_v, out_hbm.at[pl.ds(wid * _BPW, _BPW)])

    return _sc_gather


def kernel(x, e, W):
    del x
    idx3, loss = _argmin_and_loss(e.T, W)
    idx = idx3.reshape(_NW, _NCH, _CHUNK)
    quantized = _make_sc_gather()(W, idx)
    return quantized, loss[0, 0]


# speedup vs baseline: 1.3023x; 1.0586x over previous
import functools

import jax
import jax.numpy as jnp
from jax import lax
from jax.experimental import pallas as pl
from jax.experimental.pallas import tpu as pltpu
from jax.experimental.pallas import tpu_sc as plsc

N = 65536
D = 32
K = 8192

BN = 256
BK = 4096
NB_N = N // BN
NB_K = K // BK

_NW = 32
_BPW = N // _NW
_CHUNK = 128
_NCH = _BPW // _CHUNK


def _bf16_round(x):
    return x.astype(jnp.bfloat16).astype(jnp.float32)


def _vq_tc_body(et_ref, w_ref, iota_ref, idx_ref, loss_ref, bbf_ref,
                bexact_ref, bidx_ref):
    i = pl.program_id(0)
    j = pl.program_id(1)
    etb = et_ref[...]
    wb = w_ref[...]
    mm2 = lax.dot_general((wb + wb).astype(jnp.bfloat16),
                          etb.astype(jnp.bfloat16),
                          (((1,), (0,)), ((), ())),
                          preferred_element_type=jnp.float32)
    wn = jnp.sum(wb * wb, axis=1, keepdims=True)
    en = jnp.sum(etb * etb, axis=0, keepdims=True)
    s = (en + wn) - mm2
    m = jnp.min(s, axis=0, keepdims=True)
    ids = iota_ref[...]
    amin = jnp.min(jnp.where(s == m, ids, jnp.int32(K)), axis=0,
                   keepdims=True) + j * BK

    @pl.when(j == 0)
    def _():
        bbf_ref[...] = _bf16_round(m)
        bexact_ref[...] = m
        bidx_ref[...] = amin

    @pl.when(j > 0)
    def _():
        upd = m < bbf_ref[...]
        bbf_ref[...] = jnp.where(upd, _bf16_round(m), bbf_ref[...])
        bexact_ref[...] = jnp.where(upd, m, bexact_ref[...])
        bidx_ref[...] = jnp.where(upd, amin, bidx_ref[...])

    @pl.when(j == NB_K - 1)
    def _():
        idx_ref[...] = bidx_ref[...].reshape(1, 1, BN)
        part = jnp.sum(bexact_ref[...])
        prev = jnp.where(i == 0, jnp.zeros((1, 1), jnp.float32),
                         loss_ref[...])
        acc = prev + part
        scale = jnp.float32(1.25 / (N * D))
        loss_ref[...] = jnp.where(i == NB_N - 1, acc * scale, acc)


def _argmin_and_loss(eT, w):
    iota_col = jnp.arange(BK, dtype=jnp.int32).reshape(BK, 1)
    return pl.pallas_call(
        _vq_tc_body,
        grid=(NB_N, NB_K),
        in_specs=[
            pl.BlockSpec((D, BN), lambda i, j: (0, i)),
            pl.BlockSpec((BK, D), lambda i, j: (j, 0)),
            pl.BlockSpec((BK, 1), lambda i, j: (0, 0)),
        ],
        out_specs=[
            pl.BlockSpec((1, 1, BN), lambda i, j: (i, 0, 0)),
            pl.BlockSpec((1, 1), lambda i, j: (0, 0)),
        ],
        out_shape=[
            jax.ShapeDtypeStruct((NB_N, 1, BN), jnp.int32),
            jax.ShapeDtypeStruct((1, 1), jnp.float32),
        ],
        scratch_shapes=[
            pltpu.VMEM((1, BN), jnp.float32),
            pltpu.VMEM((1, BN), jnp.float32),
            pltpu.VMEM((1, BN), jnp.int32),
        ],
        compiler_params=pltpu.CompilerParams(
            dimension_semantics=("arbitrary", "arbitrary"),
        ),
    )(eT, w, iota_col)


@functools.cache
def _make_sc_gather():
    @functools.partial(
        pl.kernel,
        mesh=plsc.VectorSubcoreMesh(core_axis_name="c", subcore_axis_name="s"),
        out_type=jax.ShapeDtypeStruct((N, D), jnp.float32),
        scratch_types=[
            pltpu.VMEM((_NCH, _CHUNK), jnp.int32),
            pltpu.VMEM((_BPW, D), jnp.float32),
            pltpu.SemaphoreType.DMA,
        ],
        compiler_params=pltpu.CompilerParams(use_tc_tiling_on_sc=False),
    )
    def _sc_gather(w_hbm, idx_hbm, out_hbm, idx_v, rows_v, sem):
        wid = lax.axis_index("s") * 2 + lax.axis_index("c")
        pltpu.sync_copy(idx_hbm.at[wid], idx_v)
        copies = []
        for c in range(_NCH):
            copies.append(pltpu.async_copy(
                w_hbm.at[idx_v.at[c]],
                rows_v.at[pl.ds(c * _CHUNK, _CHUNK)],
                sem))
        for cp in copies:
            cp.wait()
        pltpu.sync_copy(rows_v, out_hbm.at[pl.ds(wid * _BPW, _BPW)])

    return _sc_gather


def kernel(x, e, W):
    del x
    idx3, loss = _argmin_and_loss(e.T, W)
    idx = idx3.reshape(_NW, _NCH, _CHUNK)
    quantized = _make_sc_gather()(W, idx)
    return quantized, loss[0, 0]


# speedup vs baseline: 1.3977x; 1.0732x over previous
import functools

import jax
import jax.numpy as jnp
from jax import lax
from jax.experimental import pallas as pl
from jax.experimental.pallas import tpu as pltpu
from jax.experimental.pallas import tpu_sc as plsc

N = 65536
D = 32
K = 8192

BN = 512
BK = 4096
NB_N = N // BN
NB_K = K // BK

_NW = 32
_BPW = N // _NW
_CHUNK = 128
_NCH = _BPW // _CHUNK


def _bf16_round(x):
    return x.astype(jnp.bfloat16).astype(jnp.float32)


def _vq_tc_body(et_ref, w_ref, iota_ref, idx_ref, loss_ref, bbf_ref,
                bexact_ref, bidx_ref):
    i = pl.program_id(0)
    j = pl.program_id(1)
    etb = et_ref[...]
    wb = w_ref[...]
    mm2 = lax.dot_general((wb + wb).astype(jnp.bfloat16),
                          etb.astype(jnp.bfloat16),
                          (((1,), (0,)), ((), ())),
                          preferred_element_type=jnp.float32)
    wn = jnp.sum(wb * wb, axis=1, keepdims=True)
    en = jnp.sum(etb * etb, axis=0, keepdims=True)
    s = (en + wn) - mm2
    m = jnp.min(s, axis=0, keepdims=True)
    ids = iota_ref[...]
    amin = jnp.min(jnp.where(s == m, ids, jnp.int32(K)), axis=0,
                   keepdims=True) + j * BK

    @pl.when(j == 0)
    def _():
        bbf_ref[...] = _bf16_round(m)
        bexact_ref[...] = m
        bidx_ref[...] = amin

    @pl.when(j > 0)
    def _():
        upd = m < bbf_ref[...]
        bbf_ref[...] = jnp.where(upd, _bf16_round(m), bbf_ref[...])
        bexact_ref[...] = jnp.where(upd, m, bexact_ref[...])
        bidx_ref[...] = jnp.where(upd, amin, bidx_ref[...])

    @pl.when(j == NB_K - 1)
    def _():
        idx_ref[...] = bidx_ref[...].reshape(1, 1, BN)
        part = jnp.sum(bexact_ref[...])
        prev = jnp.where(i == 0, jnp.zeros((1, 1), jnp.float32),
                         loss_ref[...])
        acc = prev + part
        scale = jnp.float32(1.25 / (N * D))
        loss_ref[...] = jnp.where(i == NB_N - 1, acc * scale, acc)


def _argmin_and_loss(eT, w):
    iota_col = jnp.arange(BK, dtype=jnp.int32).reshape(BK, 1)
    return pl.pallas_call(
        _vq_tc_body,
        grid=(NB_N, NB_K),
        in_specs=[
            pl.BlockSpec((D, BN), lambda i, j: (0, i)),
            pl.BlockSpec((BK, D), lambda i, j: (j, 0)),
            pl.BlockSpec((BK, 1), lambda i, j: (0, 0)),
        ],
        out_specs=[
            pl.BlockSpec((1, 1, BN), lambda i, j: (i, 0, 0)),
            pl.BlockSpec((1, 1), lambda i, j: (0, 0)),
        ],
        out_shape=[
            jax.ShapeDtypeStruct((NB_N, 1, BN), jnp.int32),
            jax.ShapeDtypeStruct((1, 1), jnp.float32),
        ],
        scratch_shapes=[
            pltpu.VMEM((1, BN), jnp.float32),
            pltpu.VMEM((1, BN), jnp.float32),
            pltpu.VMEM((1, BN), jnp.int32),
        ],
        compiler_params=pltpu.CompilerParams(
            dimension_semantics=("arbitrary", "arbitrary"),
        ),
    )(eT, w, iota_col)


@functools.cache
def _make_sc_gather():
    @functools.partial(
        pl.kernel,
        mesh=plsc.VectorSubcoreMesh(core_axis_name="c", subcore_axis_name="s"),
        out_type=jax.ShapeDtypeStruct((N, D), jnp.float32),
        scratch_types=[
            pltpu.VMEM((_NCH, _CHUNK), jnp.int32),
            pltpu.VMEM((_BPW, D), jnp.float32),
            pltpu.SemaphoreType.DMA,
        ],
        compiler_params=pltpu.CompilerParams(use_tc_tiling_on_sc=False),
    )
    def _sc_gather(w_hbm, idx_hbm, out_hbm, idx_v, rows_v, sem):
        wid = lax.axis_index("s") * 2 + lax.axis_index("c")
        pltpu.sync_copy(idx_hbm.at[wid], idx_v)
        copies = []
        for c in range(_NCH):
            copies.append(pltpu.async_copy(
                w_hbm.at[idx_v.at[c]],
                rows_v.at[pl.ds(c * _CHUNK, _CHUNK)],
                sem))
        for cp in copies:
            cp.wait()
        pltpu.sync_copy(rows_v, out_hbm.at[pl.ds(wid * _BPW, _BPW)])

    return _sc_gather


def kernel(x, e, W):
    del x
    idx3, loss = _argmin_and_loss(e.T, W)
    idx = idx3.reshape(_NW, _NCH, _CHUNK)
    quantized = _make_sc_gather()(W, idx)
    return quantized, loss[0, 0]


# speedup vs baseline: 1.5001x; 1.0733x over previous
import functools

import jax
import jax.numpy as jnp
from jax import lax
from jax.experimental import pallas as pl
from jax.experimental.pallas import tpu as pltpu
from jax.experimental.pallas import tpu_sc as plsc

N = 65536
D = 32
K = 8192

BN = 1024
BK = 4096
NB_N = N // BN
NB_K = K // BK

_NW = 32
_BPW = N // _NW
_CHUNK = 128
_NCH = _BPW // _CHUNK


def _bf16_round(x):
    return x.astype(jnp.bfloat16).astype(jnp.float32)


def _vq_tc_body(et_ref, w_ref, iota_ref, idx_ref, loss_ref, bbf_ref,
                bexact_ref, bidx_ref):
    i = pl.program_id(0)
    j = pl.program_id(1)
    etb = et_ref[...]
    wb = w_ref[...]
    mm2 = lax.dot_general((wb + wb).astype(jnp.bfloat16),
                          etb.astype(jnp.bfloat16),
                          (((1,), (0,)), ((), ())),
                          preferred_element_type=jnp.float32)
    wn = jnp.sum(wb * wb, axis=1, keepdims=True)
    en = jnp.sum(etb * etb, axis=0, keepdims=True)
    s = (en + wn) - mm2
    m = jnp.min(s, axis=0, keepdims=True)
    ids = iota_ref[...]
    amin = jnp.min(jnp.where(s == m, ids, jnp.int32(K)), axis=0,
                   keepdims=True) + j * BK

    @pl.when(j == 0)
    def _():
        bbf_ref[...] = _bf16_round(m)
        bexact_ref[...] = m
        bidx_ref[...] = amin

    @pl.when(j > 0)
    def _():
        upd = m < bbf_ref[...]
        bbf_ref[...] = jnp.where(upd, _bf16_round(m), bbf_ref[...])
        bexact_ref[...] = jnp.where(upd, m, bexact_ref[...])
        bidx_ref[...] = jnp.where(upd, amin, bidx_ref[...])

    @pl.when(j == NB_K - 1)
    def _():
        idx_ref[...] = bidx_ref[...].reshape(1, 1, BN)
        part = jnp.sum(bexact_ref[...])
        prev = jnp.where(i == 0, jnp.zeros((1, 1), jnp.float32),
                         loss_ref[...])
        acc = prev + part
        scale = jnp.float32(1.25 / (N * D))
        loss_ref[...] = jnp.where(i == NB_N - 1, acc * scale, acc)


def _argmin_and_loss(eT, w):
    iota_col = jnp.arange(BK, dtype=jnp.int32).reshape(BK, 1)
    return pl.pallas_call(
        _vq_tc_body,
        grid=(NB_N, NB_K),
        in_specs=[
            pl.BlockSpec((D, BN), lambda i, j: (0, i)),
            pl.BlockSpec((BK, D), lambda i, j: (j, 0)),
            pl.BlockSpec((BK, 1), lambda i, j: (0, 0)),
        ],
        out_specs=[
            pl.BlockSpec((1, 1, BN), lambda i, j: (i, 0, 0)),
            pl.BlockSpec((1, 1), lambda i, j: (0, 0)),
        ],
        out_shape=[
            jax.ShapeDtypeStruct((NB_N, 1, BN), jnp.int32),
            jax.ShapeDtypeStruct((1, 1), jnp.float32),
        ],
        scratch_shapes=[
            pltpu.VMEM((1, BN), jnp.float32),
            pltpu.VMEM((1, BN), jnp.float32),
            pltpu.VMEM((1, BN), jnp.int32),
        ],
        compiler_params=pltpu.CompilerParams(
            dimension_semantics=("arbitrary", "arbitrary"),
        ),
    )(eT, w, iota_col)


@functools.cache
def _make_sc_gather():
    @functools.partial(
        pl.kernel,
        mesh=plsc.VectorSubcoreMesh(core_axis_name="c", subcore_axis_name="s"),
        out_type=jax.ShapeDtypeStruct((N, D), jnp.float32),
        scratch_types=[
            pltpu.VMEM((_NCH, _CHUNK), jnp.int32),
            pltpu.VMEM((_BPW, D), jnp.float32),
            pltpu.SemaphoreType.DMA,
        ],
        compiler_params=pltpu.CompilerParams(use_tc_tiling_on_sc=False),
    )
    def _sc_gather(w_hbm, idx_hbm, out_hbm, idx_v, rows_v, sem):
        wid = lax.axis_index("s") * 2 + lax.axis_index("c")
        pltpu.sync_copy(idx_hbm.at[wid], idx_v)
        copies = []
        for c in range(_NCH):
            copies.append(pltpu.async_copy(
                w_hbm.at[idx_v.at[c]],
                rows_v.at[pl.ds(c * _CHUNK, _CHUNK)],
                sem))
        for cp in copies:
            cp.wait()
        pltpu.sync_copy(rows_v, out_hbm.at[pl.ds(wid * _BPW, _BPW)])

    return _sc_gather


def kernel(x, e, W):
    del x
    idx3, loss = _argmin_and_loss(e.T, W)
    idx = idx3.reshape(_NW, _NCH, _CHUNK)
    quantized = _make_sc_gather()(W, idx)
    return quantized, loss[0, 0]


# speedup vs baseline: 1.6466x; 1.0977x over previous
import functools

import jax
import jax.numpy as jnp
from jax import lax
from jax.experimental import pallas as pl
from jax.experimental.pallas import tpu as pltpu
from jax.experimental.pallas import tpu_sc as plsc

N = 65536
D = 32
K = 8192

BN = 2048
BK = 4096
NB_N = N // BN
NB_K = K // BK

_NW = 32
_BPW = N // _NW
_CHUNK = 128
_NCH = _BPW // _CHUNK


def _bf16_round(x):
    return x.astype(jnp.bfloat16).astype(jnp.float32)


def _vq_tc_body(et_ref, w_ref, iota_ref, idx_ref, loss_ref, bbf_ref,
                bexact_ref, bidx_ref):
    i = pl.program_id(0)
    j = pl.program_id(1)
    etb = et_ref[...]
    wb = w_ref[...]
    mm2 = lax.dot_general((wb + wb).astype(jnp.bfloat16),
                          etb.astype(jnp.bfloat16),
                          (((1,), (0,)), ((), ())),
                          preferred_element_type=jnp.float32)
    wn = jnp.sum(wb * wb, axis=1, keepdims=True)
    en = jnp.sum(etb * etb, axis=0, keepdims=True)
    s = (en + wn) - mm2
    m = jnp.min(s, axis=0, keepdims=True)
    ids = iota_ref[...]
    amin = jnp.min(jnp.where(s == m, ids, jnp.int32(K)), axis=0,
                   keepdims=True) + j * BK

    @pl.when(j == 0)
    def _():
        bbf_ref[...] = _bf16_round(m)
        bexact_ref[...] = m
        bidx_ref[...] = amin

    @pl.when(j > 0)
    def _():
        upd = m < bbf_ref[...]
        bbf_ref[...] = jnp.where(upd, _bf16_round(m), bbf_ref[...])
        bexact_ref[...] = jnp.where(upd, m, bexact_ref[...])
        bidx_ref[...] = jnp.where(upd, amin, bidx_ref[...])

    @pl.when(j == NB_K - 1)
    def _():
        idx_ref[...] = bidx_ref[...].reshape(1, 1, BN)
        part = jnp.sum(bexact_ref[...])
        prev = jnp.where(i == 0, jnp.zeros((1, 1), jnp.float32),
                         loss_ref[...])
        acc = prev + part
        scale = jnp.float32(1.25 / (N * D))
        loss_ref[...] = jnp.where(i == NB_N - 1, acc * scale, acc)


def _argmin_and_loss(eT, w):
    iota_col = jnp.arange(BK, dtype=jnp.int32).reshape(BK, 1)
    return pl.pallas_call(
        _vq_tc_body,
        grid=(NB_N, NB_K),
        in_specs=[
            pl.BlockSpec((D, BN), lambda i, j: (0, i)),
            pl.BlockSpec((BK, D), lambda i, j: (j, 0)),
            pl.BlockSpec((BK, 1), lambda i, j: (0, 0)),
        ],
        out_specs=[
            pl.BlockSpec((1, 1, BN), lambda i, j: (i, 0, 0)),
            pl.BlockSpec((1, 1), lambda i, j: (0, 0)),
        ],
        out_shape=[
            jax.ShapeDtypeStruct((NB_N, 1, BN), jnp.int32),
            jax.ShapeDtypeStruct((1, 1), jnp.float32),
        ],
        scratch_shapes=[
            pltpu.VMEM((1, BN), jnp.float32),
            pltpu.VMEM((1, BN), jnp.float32),
            pltpu.VMEM((1, BN), jnp.int32),
        ],
        compiler_params=pltpu.CompilerParams(
            dimension_semantics=("arbitrary", "arbitrary"),
        ),
    )(eT, w, iota_col)


@functools.cache
def _make_sc_gather():
    @functools.partial(
        pl.kernel,
        mesh=plsc.VectorSubcoreMesh(core_axis_name="c", subcore_axis_name="s"),
        out_type=jax.ShapeDtypeStruct((N, D), jnp.float32),
        scratch_types=[
            pltpu.VMEM((_NCH, _CHUNK), jnp.int32),
            pltpu.VMEM((_BPW, D), jnp.float32),
            pltpu.SemaphoreType.DMA,
        ],
        compiler_params=pltpu.CompilerParams(use_tc_tiling_on_sc=False),
    )
    def _sc_gather(w_hbm, idx_hbm, out_hbm, idx_v, rows_v, sem):
        wid = lax.axis_index("s") * 2 + lax.axis_index("c")
        pltpu.sync_copy(idx_hbm.at[wid], idx_v)
        copies = []
        for c in range(_NCH):
            copies.append(pltpu.async_copy(
                w_hbm.at[idx_v.at[c]],
                rows_v.at[pl.ds(c * _CHUNK, _CHUNK)],
                sem))
        for cp in copies:
            cp.wait()
        pltpu.sync_copy(rows_v, out_hbm.at[pl.ds(wid * _BPW, _BPW)])

    return _sc_gather


def kernel(x, e, W):
    del x
    idx3, loss = _argmin_and_loss(e.T, W)
    idx = idx3.reshape(_NW, _NCH, _CHUNK)
    quantized = _make_sc_gather()(W, idx)
    return quantized, loss[0, 0]
